# baseline (speedup 1.0000x reference)
import jax
import jax.numpy as jnp
from jax import lax
from jax.experimental import pallas as pl
from jax.experimental.pallas import tpu as pltpu
from jax.experimental.pallas import tpu_sc as plsc
import functools

IDX_W = 128
K = 10
MACRO = K * IDX_W


def _make_lookup(n_macros_total, n_workers, D):
    mesh = plsc.VectorSubcoreMesh(core_axis_name="c", subcore_axis_name="s")
    nc = mesh.num_cores
    per_worker = n_macros_total // n_workers
    assert per_worker % 2 == 0

    @functools.partial(
        pl.kernel,
        out_type=jax.ShapeDtypeStruct((n_macros_total, MACRO, D), jnp.float32),
        mesh=mesh,
        scratch_types=[
            pltpu.VMEM((K, IDX_W), jnp.int32),
            pltpu.VMEM((K, IDX_W), jnp.int32),
            pltpu.VMEM((MACRO, D), jnp.float32),
            pltpu.VMEM((MACRO, D), jnp.float32),
            pltpu.SemaphoreType.DMA,
            pltpu.SemaphoreType.DMA,
            pltpu.SemaphoreType.DMA,
            pltpu.SemaphoreType.DMA,
        ],
        compiler_params=pltpu.CompilerParams(use_tc_tiling_on_sc=False),
    )
    def lookup(idx_hbm, table_hbm, out_hbm, idx0, idx1, rows0, rows1,
               gsem0, gsem1, wsem0, wsem1):
        wid = lax.axis_index("s") * nc + lax.axis_index("c")
        base = wid * per_worker

        def fire_gathers(idx_v, rows_v, sem):
            return [
                pltpu.async_copy(
                    table_hbm.at[idx_v.at[j]],
                    rows_v.at[pl.ds(j * IDX_W, IDX_W)],
                    sem,
                )
                for j in range(K)
            ]

        @pl.loop(0, per_worker, step=2)
        def _(g):
            m0 = base + g
            m1 = m0 + 1
            pltpu.sync_copy(idx_hbm.at[m0], idx0)
            @pl.when(g > 0)
            def _():
                pltpu.make_async_copy(rows0, out_hbm.at[m0], wsem0).wait()
            c0 = fire_gathers(idx0, rows0, gsem0)
            pltpu.sync_copy(idx_hbm.at[m1], idx1)
            @pl.when(g > 0)
            def _():
                pltpu.make_async_copy(rows1, out_hbm.at[m1], wsem1).wait()
            c1 = fire_gathers(idx1, rows1, gsem1)
            for c in c0:
                c.wait()
            pltpu.async_copy(rows0, out_hbm.at[m0], wsem0)
            for c in c1:
                c.wait()
            pltpu.async_copy(rows1, out_hbm.at[m1], wsem1)

        last = base + per_worker - 2
        pltpu.make_async_copy(rows0, out_hbm.at[last], wsem0).wait()
        pltpu.make_async_copy(rows1, out_hbm.at[last + 1], wsem1).wait()

    return lookup


def kernel(token_ids, weight):
    B, H = token_ids.shape
    V, D = weight.shape
    total = B * H
    n_workers = 32
    assert total % (n_workers * MACRO * 2) == 0
    n_macros = total // MACRO
    idx = token_ids.astype(jnp.int32).reshape(n_macros, K, IDX_W)
    wt = lax.optimization_barrier(weight.T)
    w_rows = wt.T
    out = _make_lookup(n_macros, n_workers, D)(idx, w_rows)
    out = out.reshape(B * H, D)
    o3 = out.reshape(B, H, D)
    o4 = lax.optimization_barrier(jnp.transpose(o3, (1, 2, 0)))
    return jnp.transpose(o4, (2, 0, 1))


# speedup vs baseline: 1.2818x; 1.2818x over previous
import jax
import jax.numpy as jnp
from jax import lax
from jax.experimental import pallas as pl
from jax.experimental.pallas import tpu as pltpu
from jax.experimental.pallas import tpu_sc as plsc
import functools

IDX_W = 128
NB = 512
NJ = NB // IDX_W


def _make_lookup(B, H, D):
    mesh = plsc.VectorSubcoreMesh(core_axis_name="c", subcore_axis_name="s")
    nc = mesh.num_cores
    g_dim = D // 8
    inner = (B // IDX_W) * 8 * IDX_W
    blk = NJ * 8 * IDX_W

    @functools.partial(
        pl.kernel,
        out_type=jax.ShapeDtypeStruct((H, g_dim, inner), jnp.float32),
        mesh=mesh,
        scratch_types=[
            pltpu.VMEM((H, NB), jnp.int32),
            pltpu.VMEM((NB, D), jnp.float32),
            pltpu.VMEM((NB, D), jnp.float32),
            pltpu.VMEM((g_dim, blk), jnp.float32),
            pltpu.VMEM((g_dim, blk), jnp.float32),
            pltpu.SemaphoreType.DMA,
            pltpu.SemaphoreType.DMA,
            pltpu.SemaphoreType.DMA,
            pltpu.SemaphoreType.DMA,
        ],
        compiler_params=pltpu.CompilerParams(
            use_tc_tiling_on_sc=False, needs_layout_passes=False
        ),
    )
    def lookup(tid_hbm, table_hbm, out_hbm, idxv, rows0, rows1, st0, st1,
               gsem0, gsem1, wsem0, wsem1):
        wid = lax.axis_index("s") * nc + lax.axis_index("c")
        b0 = wid * NB
        f0 = wid * blk
        i16 = lax.iota(jnp.int32, 16)

        pltpu.sync_copy(tid_hbm.at[:, pl.ds(b0, NB)], idxv)

        def fire(h, rows, sem):
            return [
                pltpu.async_copy(
                    table_hbm.at[idxv.at[h, pl.ds(j * IDX_W, IDX_W)]],
                    rows.at[pl.ds(j * IDX_W, IDX_W)],
                    sem,
                )
                for j in range(NJ)
            ]

        def transpose(rows, st):
            @pl.loop(0, g_dim * NJ * 8, unroll=4)
            def _(q):
                s_ = q & 7
                j_ = (q >> 3) & (NJ - 1)
                g_ = q >> 5
                col = jnp.broadcast_to(8 * g_ + s_, (16,))
                riv = i16 + jnp.broadcast_to(j_ * IDX_W, (16,))
                base = j_ * (8 * IDX_W) + s_ * IDX_W
                for l0 in range(0, IDX_W, 16):
                    v = plsc.load_gather(rows, [riv + l0, col])
                    st[g_, pl.ds(base + l0, 16)] = v

        def writeback(h, st, sem):
            for g_ in range(g_dim):
                pltpu.async_copy(
                    st.at[g_], out_hbm.at[h, g_, pl.ds(f0, blk)], sem
                )

        def drain_wb(st, sem):
            pltpu.make_async_copy(st, out_hbm.at[0, :, pl.ds(f0, blk)], sem).wait()

        @pl.loop(0, H, step=2)
        def _(g):
            c0 = fire(g, rows0, gsem0)
            @pl.when(g > 2)
            def _():
                drain_wb(st1, wsem1)
            @pl.when(g > 0)
            def _():
                transpose(rows1, st1)
                writeback(g - 1, st1, wsem1)
            for c in c0:
                c.wait()
            c1 = fire(g + 1, rows1, gsem1)
            @pl.when(g > 0)
            def _():
                drain_wb(st0, wsem0)
            transpose(rows0, st0)
            writeback(g, st0, wsem0)
            for c in c1:
                c.wait()

        drain_wb(st1, wsem1)
        transpose(rows1, st1)
        writeback(H - 1, st1, wsem1)
        drain_wb(st0, wsem0)
        drain_wb(st1, wsem1)

    return lookup


def kernel(token_ids, weight):
    B, H = token_ids.shape
    V, D = weight.shape
    tid_t = token_ids.astype(jnp.int32).T
    o = _make_lookup(B, H, D)(tid_t, weight)
    o5 = o.reshape(H, D // 8, B // IDX_W, 8, IDX_W)
    t1 = jnp.transpose(o5, (2, 4, 0, 1, 3))
    return t1.reshape(B, H, D)
